# baseline (speedup 1.0000x reference)
import functools
import math

import jax
import jax.numpy as jnp
from jax import lax
from jax.experimental import pallas as pl
from jax.experimental.pallas import tpu as pltpu
from jax.experimental.pallas import tpu_sc as plsc

_B = 4096
_L = 50
_LP = 56
_DIM = 128
_SCALE = math.sqrt(_DIM)

_K = 4
_BS = _B // _K

_info = plsc.get_sparse_core_info()
_NC = _info.num_cores
_NS = _info.num_subcores
_NW = _NC * _NS
_BPW = _BS // _NW
_NBUF = 8
_LA = 4

_mesh = plsc.VectorSubcoreMesh(core_axis_name="c", subcore_axis_name="s")


def _make_sc_stage(stage):
    @functools.partial(
        pl.kernel,
        mesh=_mesh,
        out_type=jax.ShapeDtypeStruct((_BS, _LP, _DIM), jnp.float32),
        scratch_types=[
            pltpu.VMEM((_BPW, _L), jnp.int32),
            pltpu.VMEM((_NBUF, _LP, _DIM), jnp.float32),
        ]
        + [pltpu.SemaphoreType.DMA] * (2 * _NBUF),
    )
    def sc_stage(table_hbm, idx_hbm, out_hbm, idx_v, rows_v, *sems):
        sem_g = sems[:_NBUF]
        sem_s = sems[_NBUF:]
        wid = lax.axis_index("s") * _NC + lax.axis_index("c")
        base = stage * _BS + wid * _BPW
        obase = wid * _BPW
        pltpu.sync_copy(idx_hbm.at[pl.ds(base, _BPW)], idx_v)

        def gather_copy(g, b):
            return pltpu.make_async_copy(
                table_hbm.at[idx_v.at[g]],
                rows_v.at[b, pl.ds(0, _L)],
                sem_g[b],
            )

        def scatter_copy(g, b):
            return pltpu.make_async_copy(
                rows_v.at[b], out_hbm.at[obase + g], sem_s[b]
            )

        for g in range(_LA):
            gather_copy(g, g).start()

        def round_body(r, carry):
            for b in range(_NBUF):
                g = r * _NBUF + b
                gather_copy(g, b).wait()
                scatter_copy(g, b).start()
                bf = (b + _LA) % _NBUF
                gf = g + _LA

                @pl.when(gf < _BPW)
                def _():
                    @pl.when(gf >= _NBUF)
                    def _():
                        scatter_copy(gf - _NBUF, bf).wait()

                    gather_copy(gf, bf).start()

            return carry

        lax.fori_loop(0, _BPW // _NBUF, round_body, 0)

        for b in range(_NBUF):
            scatter_copy(_BPW - _NBUF + b, b).wait()

    return sc_stage


_GB = 8
_GL = 8


def _tc_body_first(flat_ref, out_ref):
    out_ref[...] = flat_ref[...] * _SCALE


def _tc_body_rest(buf_ref, flat_ref, out_ref):
    del buf_ref
    out_ref[...] = flat_ref[...] * _SCALE


def _make_tc_stage(stage):
    grid = (_BS // _GB, (_L + _GL - 1) // _GL)
    out_spec = pl.BlockSpec(
        (_GB, _GL, _DIM),
        lambda i, j, _s=stage: (_s * (_BS // _GB) + i, j, 0),
    )
    flat_spec = pl.BlockSpec((_GB, _GL, _DIM), lambda i, j: (i, j, 0))
    out_shape = jax.ShapeDtypeStruct((_B, _L, _DIM), jnp.float32)
    if stage == 0:
        return pl.pallas_call(
            _tc_body_first,
            grid=grid,
            in_specs=[flat_spec],
            out_specs=out_spec,
            out_shape=out_shape,
        )
    return pl.pallas_call(
        _tc_body_rest,
        grid=grid,
        in_specs=[pl.BlockSpec(memory_space=pl.ANY), flat_spec],
        out_specs=out_spec,
        out_shape=out_shape,
        input_output_aliases={0: 0},
    )


_SC_STAGES = [_make_sc_stage(s) for s in range(_K)]
_TC_STAGES = [_make_tc_stage(s) for s in range(_K)]


def kernel(x, table):
    flats = [_SC_STAGES[s](table, x) for s in range(_K)]
    buf = _TC_STAGES[0](flats[0])
    for s in range(1, _K):
        buf = _TC_STAGES[s](buf, flats[s])
    return buf


# speedup vs baseline: 7.6117x; 7.6117x over previous
import functools
import math

import jax
import jax.numpy as jnp
from jax import lax
from jax.experimental import pallas as pl
from jax.experimental.pallas import tpu as pltpu
from jax.experimental.pallas import tpu_sc as plsc

_B = 4096
_L = 50
_LP = 56
_DIM = 128
_SCALE = math.sqrt(_DIM)

_K = 4
_BS = _B // _K

_info = plsc.get_sparse_core_info()
_NC = _info.num_cores
_NS = _info.num_subcores
_NW = _NC * _NS
_BPW = _BS // _NW
_NBUF = 8
_LA = 4

_mesh = plsc.VectorSubcoreMesh(core_axis_name="c", subcore_axis_name="s")


def _make_sc_stage(stage):
    @functools.partial(
        pl.kernel,
        mesh=_mesh,
        out_type=jax.ShapeDtypeStruct((_BS, _LP, _DIM), jnp.float32),
        scratch_types=[
            pltpu.VMEM((_BPW, _L), jnp.int32),
            pltpu.VMEM((_NBUF, _LP, _DIM), jnp.float32),
        ]
        + [pltpu.SemaphoreType.DMA] * (2 * _NBUF),
    )
    def sc_stage(table_hbm, idx_hbm, out_hbm, idx_v, rows_v, *sems):
        sem_g = sems[:_NBUF]
        sem_s = sems[_NBUF:]
        wid = lax.axis_index("s") * _NC + lax.axis_index("c")
        base = stage * _BS + wid * _BPW
        obase = wid * _BPW
        pltpu.sync_copy(idx_hbm.at[pl.ds(base, _BPW)], idx_v)

        def gather_copy(g, b):
            return pltpu.make_async_copy(
                table_hbm.at[idx_v.at[g]],
                rows_v.at[b, pl.ds(0, _L)],
                sem_g[b],
            )

        def scatter_copy(g, b):
            return pltpu.make_async_copy(
                rows_v.at[b], out_hbm.at[obase + g], sem_s[b]
            )

        for g in range(_LA):
            gather_copy(g, g).start()

        def round_body(r, carry):
            for b in range(_NBUF):
                g = r * _NBUF + b
                gather_copy(g, b).wait()
                scatter_copy(g, b).start()
                bf = (b + _LA) % _NBUF
                gf = g + _LA

                @pl.when(gf < _BPW)
                def _():
                    @pl.when(gf >= _NBUF)
                    def _():
                        scatter_copy(gf - _NBUF, bf).wait()

                    gather_copy(gf, bf).start()

            return carry

        lax.fori_loop(0, _BPW // _NBUF, round_body, 0)

        for b in range(_NBUF):
            scatter_copy(_BPW - _NBUF + b, b).wait()

    return sc_stage


_GB = 64


def _tc_body_first(flat_ref, out_ref):
    out_ref[...] = flat_ref[:, : _L, :] * _SCALE


def _tc_body_rest(buf_ref, flat_ref, out_ref):
    del buf_ref
    out_ref[...] = flat_ref[:, : _L, :] * _SCALE


def _make_tc_stage(stage):
    grid = (_BS // _GB,)
    out_spec = pl.BlockSpec(
        (_GB, _L, _DIM),
        lambda i, _s=stage: (_s * (_BS // _GB) + i, 0, 0),
    )
    flat_spec = pl.BlockSpec((_GB, _LP, _DIM), lambda i: (i, 0, 0))
    out_shape = jax.ShapeDtypeStruct((_B, _L, _DIM), jnp.float32)
    if stage == 0:
        return pl.pallas_call(
            _tc_body_first,
            grid=grid,
            in_specs=[flat_spec],
            out_specs=out_spec,
            out_shape=out_shape,
        )
    return pl.pallas_call(
        _tc_body_rest,
        grid=grid,
        in_specs=[pl.BlockSpec(memory_space=pl.ANY), flat_spec],
        out_specs=out_spec,
        out_shape=out_shape,
        input_output_aliases={0: 0},
    )


_SC_STAGES = [_make_sc_stage(s) for s in range(_K)]
_TC_STAGES = [_make_tc_stage(s) for s in range(_K)]


def kernel(x, table):
    flats = [_SC_STAGES[s](table, x) for s in range(_K)]
    buf = _TC_STAGES[0](flats[0])
    for s in range(1, _K):
        buf = _TC_STAGES[s](buf, flats[s])
    return buf
